# baseline (speedup 1.0000x reference)
import functools

import jax
import jax.numpy as jnp
from jax import lax
from jax.experimental import pallas as pl
from jax.experimental.pallas import tpu as pltpu
from jax.experimental.pallas import tpu_sc as plsc

N, F, E = 10000, 128, 320000
B, A, H = 256, 10, 128

NC, NS, L = 2, 16, 16
RPT = 320
RPC = NS * RPT
NPAD = NC * RPC
BACC = B + 16
CHUNK = 64
NCH = RPT // CHUNK
GPT = B // NS
CUR_PT = B // (NC * NS)


def _elu(x):
    return jnp.where(x > 0, x, jnp.exp(x) - 1.0)


def _encoder_kernel(nf_ref, w1_ref, b1_ref, w2_ref, b2_ref, w3_ref, b3_ref,
                    h_ref):
    f32 = jnp.float32
    x = nf_ref[...]
    h = _elu(jnp.dot(x, w1_ref[...], preferred_element_type=f32) + b1_ref[...])
    h = _elu(jnp.dot(h, w2_ref[...], preferred_element_type=f32) + b2_ref[...])
    h = _elu(jnp.dot(h, w3_ref[...], preferred_element_type=f32) + b3_ref[...])
    h_ref[...] = h


def _sc_pool_body(h_hbm, batch2d_hbm, batch1d_hbm, cur_hbm,
                  out_sum, out_max, out_cur,
                  rows_v, idx_v, idx_flat, acc_max,
                  cur_idx, cur_rows, mbuf, macc,
                  shared_sum, shared_max, sem):
    c = lax.axis_index("c")
    s = lax.axis_index("s")
    wid = c * NS + s
    row0 = c * RPC + s * RPT
    n = jnp.clip(N - row0, 0, RPT)

    neg16 = jnp.full((L,), -jnp.inf, dtype=jnp.float32)
    zero16 = jnp.zeros((L,), dtype=jnp.float32)

    def init_acc(i, carry):
        for j in range(F // L):
            acc_max[i, pl.ds(L * j, L)] = neg16
        return carry
    lax.fori_loop(0, BACC, init_acc, 0)

    def init_z(i, carry):
        for j in range(F // L):
            mbuf[i, pl.ds(L * j, L)] = zero16
        return carry
    lax.fori_loop(0, GPT, init_z, 0)

    pltpu.sync_copy(mbuf, shared_sum.at[pl.ds(GPT * s, GPT)])

    @pl.when(s == 0)
    def _zero_pad_rows():
        pltpu.sync_copy(mbuf, shared_sum.at[pl.ds(B, BACC - B)])

    pltpu.sync_copy(h_hbm.at[pl.ds(row0, RPT)], rows_v)
    pltpu.sync_copy(batch2d_hbm.at[wid], idx_v)
    pltpu.sync_copy(batch1d_hbm.at[pl.ds(row0, RPT)], idx_flat)

    plsc.subcore_barrier()

    for k in range(NCH):
        pltpu.sync_copy(rows_v.at[pl.ds(CHUNK * k, CHUNK)],
                        shared_sum.at[idx_v.at[k]], add=True)

    def max_body(g, carry):
        bvec = idx_flat[pl.ds(L * g, L)]
        for lane in range(L):
            b = bvec[lane]
            r = L * g + lane
            for j in range(F // L):
                sl = pl.ds(L * j, L)
                acc_max[b, sl] = jnp.maximum(acc_max[b, sl], rows_v[r, sl])
        return carry
    lax.fori_loop(0, n // L, max_body, 0)

    pltpu.sync_copy(acc_max.at[pl.ds(0, B)], shared_max.at[s])

    pltpu.sync_copy(cur_hbm.at[pl.ds(wid * CUR_PT, CUR_PT)], cur_idx)
    pltpu.async_copy(h_hbm.at[cur_idx], cur_rows, sem).wait()
    pltpu.sync_copy(cur_rows, out_cur.at[pl.ds(wid * CUR_PT, CUR_PT)])

    plsc.subcore_barrier()

    seg0 = GPT * s
    pltpu.sync_copy(shared_max.at[0, pl.ds(seg0, GPT)], macc)
    for t in range(1, NS):
        pltpu.sync_copy(shared_max.at[t, pl.ds(seg0, GPT)], mbuf)

        def merge_body(i, carry):
            for j in range(F // L):
                sl = pl.ds(L * j, L)
                macc[i, sl] = jnp.maximum(macc[i, sl], mbuf[i, sl])
            return carry
        lax.fori_loop(0, GPT, merge_body, 0)

    pltpu.sync_copy(macc, out_max.at[c, pl.ds(seg0, GPT)])
    pltpu.sync_copy(shared_sum.at[pl.ds(seg0, GPT)],
                    out_sum.at[c, pl.ds(seg0, GPT)])


_sc_pool = functools.partial(
    pl.kernel,
    out_type=(
        jax.ShapeDtypeStruct((NC, B, F), jnp.float32),
        jax.ShapeDtypeStruct((NC, B, F), jnp.float32),
        jax.ShapeDtypeStruct((B, F), jnp.float32),
    ),
    mesh=plsc.VectorSubcoreMesh(core_axis_name="c", subcore_axis_name="s",
                                num_cores=NC, num_subcores=NS),
    scratch_types=[
        pltpu.VMEM((RPT, F), jnp.float32),
        pltpu.VMEM((NCH, CHUNK), jnp.int32),
        pltpu.VMEM((RPT,), jnp.int32),
        pltpu.VMEM((BACC, F), jnp.float32),
        pltpu.VMEM((CUR_PT,), jnp.int32),
        pltpu.VMEM((CUR_PT, F), jnp.float32),
        pltpu.VMEM((GPT, F), jnp.float32),
        pltpu.VMEM((GPT, F), jnp.float32),
        pltpu.VMEM_SHARED((BACC, F), jnp.float32),
        pltpu.VMEM_SHARED((NS, B, F), jnp.float32),
        pltpu.SemaphoreType.DMA,
    ],
)(_sc_pool_body)


def _heads_kernel(sum_ref, max_ref, batch_ref, cur_ref, mask_ref,
                  wa1_ref, ba1_ref, wa2_ref, ba2_ref,
                  wc1_ref, bc1_ref, wc2_ref, bc2_ref,
                  probs_ref, values_ref):
    f32 = jnp.float32
    seg_sum = sum_ref[0] + sum_ref[1]
    max_p = jnp.maximum(max_ref[0], max_ref[1])
    CC = 2000
    seg_iota = lax.broadcasted_iota(jnp.int32, (B, CC), 0)
    counts = jnp.zeros((B, 1), dtype=f32)
    for off in range(0, N, CC):
        onehot = (seg_iota == batch_ref[:, off:off + CC]).astype(f32)
        counts = counts + jnp.sum(onehot, axis=1, keepdims=True)
    mean_p = seg_sum / jnp.maximum(counts, 1.0)
    max_p = jnp.where(counts > 0, max_p, -jnp.inf)

    graph_emb = jnp.concatenate([mean_p, max_p], axis=-1)
    actor_in = jnp.concatenate([graph_emb, cur_ref[...]], axis=-1)
    a = _elu(jnp.dot(actor_in, wa1_ref[...], preferred_element_type=f32)
             + ba1_ref[...])
    logits = jnp.dot(a, wa2_ref[...], preferred_element_type=f32) + ba2_ref[...]

    amask = mask_ref[...]
    has_valid = jnp.sum(amask, axis=-1, keepdims=True) > 0
    safe_mask = jnp.where(has_valid, amask, jnp.ones_like(amask))
    logits = jnp.where(safe_mask == 0, -jnp.inf, logits)
    m = jnp.max(logits, axis=-1, keepdims=True)
    e = jnp.exp(logits - m)
    probs = e / jnp.sum(e, axis=-1, keepdims=True)
    nan_mask = jnp.any(jnp.isnan(probs), axis=-1, keepdims=True)
    probs_ref[...] = jnp.where(nan_mask, jnp.full_like(probs, 1.0 / A), probs)

    c = _elu(jnp.dot(graph_emb, wc1_ref[...], preferred_element_type=f32)
             + bc1_ref[...])
    values_ref[...] = (jnp.dot(c, wc2_ref[...], preferred_element_type=f32)
                       + bc2_ref[...])


@jax.jit
def _run(node_features, action_mask, current_node, batch,
         W1, b1, W2, b2, W3, b3, Wa1, ba1, Wa2, ba2, Wc1, bc1, Wc2, bc2):
    f32 = jnp.float32
    nf_pad = jnp.zeros((NPAD, F), dtype=f32).at[:N].set(node_features)
    batch_pad = jnp.full((NPAD,), B, dtype=jnp.int32).at[:N].set(
        batch.astype(jnp.int32))
    batch2d = batch_pad.reshape(NC * NS, NCH, CHUNK)
    cur = current_node.astype(jnp.int32)

    h = pl.pallas_call(
        _encoder_kernel,
        out_shape=jax.ShapeDtypeStruct((NPAD, F), f32),
    )(nf_pad, W1, b1.reshape(1, H), W2, b2.reshape(1, H), W3, b3.reshape(1, H))

    seg_sum, seg_max, cur_emb = _sc_pool(h, batch2d, batch_pad, cur)

    return pl.pallas_call(
        _heads_kernel,
        out_shape=(jax.ShapeDtypeStruct((B, A), f32),
                   jax.ShapeDtypeStruct((B, 1), f32)),
    )(seg_sum, seg_max, batch.astype(jnp.int32).reshape(1, N), cur_emb,
      action_mask,
      Wa1, ba1.reshape(1, 256), Wa2, ba2.reshape(1, A),
      Wc1, bc1.reshape(1, 256), Wc2, bc2.reshape(1, 1))


def kernel(node_features, edge_index, edge_features, action_mask, current_node,
           batch, W1, b1, W2, b2, W3, b3, Wa1, ba1, Wa2, ba2, Wc1, bc1,
           Wc2, bc2):
    del edge_index, edge_features
    return _run(node_features, action_mask, current_node, batch,
                W1, b1, W2, b2, W3, b3, Wa1, ba1, Wa2, ba2,
                Wc1, bc1, Wc2, bc2)


# speedup vs baseline: 1.0013x; 1.0013x over previous
import functools

import jax
import jax.numpy as jnp
from jax import lax
from jax.experimental import pallas as pl
from jax.experimental.pallas import tpu as pltpu
from jax.experimental.pallas import tpu_sc as plsc

N, F, E = 10000, 128, 320000
B, A, H = 256, 10, 128

NC, NS, L = 2, 16, 16
RPT = 320
RPC = NS * RPT
NPAD = NC * RPC
BACC = B + 16
CHUNK = 64
NCH = RPT // CHUNK
GPT = B // NS
CUR_PT = B // (NC * NS)


def _elu(x):
    return jnp.where(x > 0, x, jnp.exp(x) - 1.0)


def _encoder_kernel(nf_ref, w1_ref, b1_ref, w2_ref, b2_ref, w3_ref, b3_ref,
                    h_ref):
    f32 = jnp.float32
    x = nf_ref[...]
    h = _elu(jnp.dot(x, w1_ref[...], preferred_element_type=f32) + b1_ref[...])
    h = _elu(jnp.dot(h, w2_ref[...], preferred_element_type=f32) + b2_ref[...])
    h = _elu(jnp.dot(h, w3_ref[...], preferred_element_type=f32) + b3_ref[...])
    h_ref[...] = h


def _sc_pool_body(h_hbm, batch2d_hbm, batch1d_hbm, cur_hbm,
                  out_sum, out_max, out_cur,
                  rows_v, idx_v, idx_flat, acc_max,
                  cur_idx, cur_rows, mbuf, macc,
                  shared_sum, shared_max, sem):
    c = lax.axis_index("c")
    s = lax.axis_index("s")
    wid = c * NS + s
    row0 = c * RPC + s * RPT
    n = jnp.clip(N - row0, 0, RPT)

    neg16 = jnp.full((L,), -jnp.inf, dtype=jnp.float32)
    zero16 = jnp.zeros((L,), dtype=jnp.float32)

    def init_acc(i, carry):
        for j in range(F // L):
            acc_max[i, pl.ds(L * j, L)] = neg16
        return carry
    lax.fori_loop(0, BACC, init_acc, 0)

    def init_z(i, carry):
        for j in range(F // L):
            mbuf[i, pl.ds(L * j, L)] = zero16
        return carry
    lax.fori_loop(0, GPT, init_z, 0)

    pltpu.sync_copy(mbuf, shared_sum.at[pl.ds(GPT * s, GPT)])

    @pl.when(s == 0)
    def _zero_pad_rows():
        pltpu.sync_copy(mbuf, shared_sum.at[pl.ds(B, BACC - B)])

    pltpu.sync_copy(h_hbm.at[pl.ds(row0, RPT)], rows_v)
    pltpu.sync_copy(batch2d_hbm.at[wid], idx_v)
    pltpu.sync_copy(batch1d_hbm.at[pl.ds(row0, RPT)], idx_flat)

    plsc.subcore_barrier()

    for k in range(NCH):
        pltpu.sync_copy(rows_v.at[pl.ds(CHUNK * k, CHUNK)],
                        shared_sum.at[idx_v.at[k]], add=True)

    def max_group(g, state):
        prev, regs = state
        bvec = idx_flat[pl.ds(L * g, L)]
        for lane in range(L):
            b = bvec[lane]
            r = L * g + lane
            pen = jnp.where(b != prev, -jnp.inf, 0.0).astype(jnp.float32)
            pen_vec = jnp.broadcast_to(pen, (L,))
            new_regs = []
            for j in range(F // L):
                row = rows_v[r, pl.ds(L * j, L)]
                v = jnp.maximum(regs[j] + pen_vec, row)
                acc_max[b, pl.ds(L * j, L)] = v
                new_regs.append(v)
            regs = tuple(new_regs)
            prev = b
        return (prev, regs)

    lax.fori_loop(
        0, n // L, max_group,
        (jnp.int32(-1), tuple(neg16 for _ in range(F // L))))

    pltpu.sync_copy(acc_max.at[pl.ds(0, B)], shared_max.at[s])

    pltpu.sync_copy(cur_hbm.at[pl.ds(wid * CUR_PT, CUR_PT)], cur_idx)
    pltpu.async_copy(h_hbm.at[cur_idx], cur_rows, sem).wait()
    pltpu.sync_copy(cur_rows, out_cur.at[pl.ds(wid * CUR_PT, CUR_PT)])

    plsc.subcore_barrier()

    seg0 = GPT * s
    pltpu.sync_copy(shared_max.at[0, pl.ds(seg0, GPT)], macc)
    for t in range(1, NS):
        pltpu.sync_copy(shared_max.at[t, pl.ds(seg0, GPT)], mbuf)

        def merge_body(i, carry):
            for j in range(F // L):
                sl = pl.ds(L * j, L)
                macc[i, sl] = jnp.maximum(macc[i, sl], mbuf[i, sl])
            return carry
        lax.fori_loop(0, GPT, merge_body, 0)

    pltpu.sync_copy(macc, out_max.at[c, pl.ds(seg0, GPT)])
    pltpu.sync_copy(shared_sum.at[pl.ds(seg0, GPT)],
                    out_sum.at[c, pl.ds(seg0, GPT)])


_sc_pool = functools.partial(
    pl.kernel,
    out_type=(
        jax.ShapeDtypeStruct((NC, B, F), jnp.float32),
        jax.ShapeDtypeStruct((NC, B, F), jnp.float32),
        jax.ShapeDtypeStruct((B, F), jnp.float32),
    ),
    mesh=plsc.VectorSubcoreMesh(core_axis_name="c", subcore_axis_name="s",
                                num_cores=NC, num_subcores=NS),
    scratch_types=[
        pltpu.VMEM((RPT, F), jnp.float32),
        pltpu.VMEM((NCH, CHUNK), jnp.int32),
        pltpu.VMEM((RPT,), jnp.int32),
        pltpu.VMEM((BACC, F), jnp.float32),
        pltpu.VMEM((CUR_PT,), jnp.int32),
        pltpu.VMEM((CUR_PT, F), jnp.float32),
        pltpu.VMEM((GPT, F), jnp.float32),
        pltpu.VMEM((GPT, F), jnp.float32),
        pltpu.VMEM_SHARED((BACC, F), jnp.float32),
        pltpu.VMEM_SHARED((NS, B, F), jnp.float32),
        pltpu.SemaphoreType.DMA,
    ],
)(_sc_pool_body)


def _heads_kernel(sum_ref, max_ref, batch_ref, cur_ref, mask_ref,
                  wa1_ref, ba1_ref, wa2_ref, ba2_ref,
                  wc1_ref, bc1_ref, wc2_ref, bc2_ref,
                  probs_ref, values_ref):
    f32 = jnp.float32
    seg_sum = sum_ref[0] + sum_ref[1]
    max_p = jnp.maximum(max_ref[0], max_ref[1])
    CC = 2000
    seg_iota = lax.broadcasted_iota(jnp.int32, (B, CC), 0)
    counts = jnp.zeros((B, 1), dtype=f32)
    for off in range(0, N, CC):
        onehot = (seg_iota == batch_ref[:, off:off + CC]).astype(f32)
        counts = counts + jnp.sum(onehot, axis=1, keepdims=True)
    mean_p = seg_sum / jnp.maximum(counts, 1.0)
    max_p = jnp.where(counts > 0, max_p, -jnp.inf)

    graph_emb = jnp.concatenate([mean_p, max_p], axis=-1)
    actor_in = jnp.concatenate([graph_emb, cur_ref[...]], axis=-1)
    a = _elu(jnp.dot(actor_in, wa1_ref[...], preferred_element_type=f32)
             + ba1_ref[...])
    logits = jnp.dot(a, wa2_ref[...], preferred_element_type=f32) + ba2_ref[...]

    amask = mask_ref[...]
    has_valid = jnp.sum(amask, axis=-1, keepdims=True) > 0
    safe_mask = jnp.where(has_valid, amask, jnp.ones_like(amask))
    logits = jnp.where(safe_mask == 0, -jnp.inf, logits)
    m = jnp.max(logits, axis=-1, keepdims=True)
    e = jnp.exp(logits - m)
    probs = e / jnp.sum(e, axis=-1, keepdims=True)
    nan_mask = jnp.any(jnp.isnan(probs), axis=-1, keepdims=True)
    probs_ref[...] = jnp.where(nan_mask, jnp.full_like(probs, 1.0 / A), probs)

    c = _elu(jnp.dot(graph_emb, wc1_ref[...], preferred_element_type=f32)
             + bc1_ref[...])
    values_ref[...] = (jnp.dot(c, wc2_ref[...], preferred_element_type=f32)
                       + bc2_ref[...])


@jax.jit
def _run(node_features, action_mask, current_node, batch,
         W1, b1, W2, b2, W3, b3, Wa1, ba1, Wa2, ba2, Wc1, bc1, Wc2, bc2):
    f32 = jnp.float32
    nf_pad = jnp.zeros((NPAD, F), dtype=f32).at[:N].set(node_features)
    batch_pad = jnp.full((NPAD,), B, dtype=jnp.int32).at[:N].set(
        batch.astype(jnp.int32))
    batch2d = batch_pad.reshape(NC * NS, NCH, CHUNK)
    cur = current_node.astype(jnp.int32)

    h = pl.pallas_call(
        _encoder_kernel,
        out_shape=jax.ShapeDtypeStruct((NPAD, F), f32),
    )(nf_pad, W1, b1.reshape(1, H), W2, b2.reshape(1, H), W3, b3.reshape(1, H))

    seg_sum, seg_max, cur_emb = _sc_pool(h, batch2d, batch_pad, cur)

    return pl.pallas_call(
        _heads_kernel,
        out_shape=(jax.ShapeDtypeStruct((B, A), f32),
                   jax.ShapeDtypeStruct((B, 1), f32)),
    )(seg_sum, seg_max, batch.astype(jnp.int32).reshape(1, N), cur_emb,
      action_mask,
      Wa1, ba1.reshape(1, 256), Wa2, ba2.reshape(1, A),
      Wc1, bc1.reshape(1, 256), Wc2, bc2.reshape(1, 1))


def kernel(node_features, edge_index, edge_features, action_mask, current_node,
           batch, W1, b1, W2, b2, W3, b3, Wa1, ba1, Wa2, ba2, Wc1, bc1,
           Wc2, bc2):
    del edge_index, edge_features
    return _run(node_features, action_mask, current_node, batch,
                W1, b1, W2, b2, W3, b3, Wa1, ba1, Wa2, ba2,
                Wc1, bc1, Wc2, bc2)
